# baseline (device time: 36195 ns/iter reference)
import jax
import jax.numpy as jnp
from jax import lax
from jax.experimental import pallas as pl
from jax.experimental.pallas import tpu as pltpu

M = 2048
N = 1024
H = N // 2
CH = 64
KX = 11
KZ = 10


def kernel(x):
    def body(x_ref, out_ref, xloc, xrecv, lsem, sx, rx, sy, ry, sz, rz):
        my_x = lax.axis_index("x")
        my_y = lax.axis_index("y")
        my_z = lax.axis_index("z")
        x_peer = (1 - my_x, my_y, my_z)
        y_nbr = (my_x, 1 - my_y, my_z)
        z_nbr = (my_x, my_y, 1 - my_z)

        e_row = (22 * my_z + 5 * my_y) * CH
        c_row = (10 + 6 * my_y) * CH
        e2_row = (22 * my_z + 5 * (1 - my_y)) * CH
        c2_row = (10 + 6 * (1 - my_y)) * CH

        def unit_row(k):
            return e_row + k * CH if k < 5 else c_row + (k - 5) * CH

        def unit_row_nbr(k):
            return e2_row + k * CH if k < 5 else c2_row + (k - 5) * CH

        def run(my_lo, peer_lo):
            ld = []
            for k in range(KX):
                rows = pl.ds(unit_row(k), CH)
                d = pltpu.make_async_copy(
                    x_ref.at[0, rows, my_lo : my_lo + H], xloc.at[k], lsem.at[k]
                )
                d.start()
                ld.append(d)

            barrier_sem = pltpu.get_barrier_semaphore()
            for nbr in (x_peer, y_nbr, z_nbr):
                pl.semaphore_signal(
                    barrier_sem,
                    inc=1,
                    device_id=nbr,
                    device_id_type=pl.DeviceIdType.MESH,
                )
            pl.semaphore_wait(barrier_sem, 3)

            xr = []
            for k in range(KX):
                rows = pl.ds(unit_row(k), CH)
                d = pltpu.make_async_remote_copy(
                    src_ref=x_ref.at[0, rows, peer_lo : peer_lo + H],
                    dst_ref=xrecv.at[k],
                    send_sem=sx.at[k],
                    recv_sem=rx.at[k],
                    device_id=x_peer,
                    device_id_type=pl.DeviceIdType.MESH,
                )
                d.start()
                xr.append(d)

            yr = []
            zr = []
            for k in range(KX):
                rows = pl.ds(unit_row(k), CH)
                xr[k].wait_recv()
                ld[k].wait()
                out_ref[rows, :] = xloc[k] + xrecv[k]
                dy = pltpu.make_async_remote_copy(
                    src_ref=out_ref.at[rows],
                    dst_ref=out_ref.at[rows],
                    send_sem=sy.at[k],
                    recv_sem=ry.at[k],
                    device_id=y_nbr,
                    device_id_type=pl.DeviceIdType.MESH,
                )
                dy.start()
                yr.append(dy)
                if k < 5:
                    dz = pltpu.make_async_remote_copy(
                        src_ref=out_ref.at[rows],
                        dst_ref=out_ref.at[rows],
                        send_sem=sz.at[k],
                        recv_sem=rz.at[k],
                        device_id=z_nbr,
                        device_id_type=pl.DeviceIdType.MESH,
                    )
                    dz.start()
                    zr.append(dz)

            for k in range(KX):
                yr[k].wait_recv()
                if k < 5:
                    rows = pl.ds(unit_row_nbr(k), CH)
                    dz = pltpu.make_async_remote_copy(
                        src_ref=out_ref.at[rows],
                        dst_ref=out_ref.at[rows],
                        send_sem=sz.at[5 + k],
                        recv_sem=rz.at[5 + k],
                        device_id=z_nbr,
                        device_id_type=pl.DeviceIdType.MESH,
                    )
                    dz.start()
                    zr.append(dz)

            for j in range(KZ):
                zr[j].wait_recv()
            for k in range(KX):
                xr[k].wait_send()
                yr[k].wait_send()
            for j in range(KZ):
                zr[j].wait_send()

        @pl.when(my_x == 0)
        def _():
            run(0, H)

        @pl.when(my_x == 1)
        def _():
            run(H, 0)

    return pl.pallas_call(
        body,
        out_shape=jax.ShapeDtypeStruct((M, H), jnp.float32),
        in_specs=[pl.BlockSpec(memory_space=pl.ANY)],
        out_specs=pl.BlockSpec(memory_space=pltpu.VMEM),
        scratch_shapes=[
            pltpu.VMEM((KX, CH, H), jnp.float32),
            pltpu.VMEM((KX, CH, H), jnp.float32),
            pltpu.SemaphoreType.DMA((KX,)),
            pltpu.SemaphoreType.DMA((KX,)),
            pltpu.SemaphoreType.DMA((KX,)),
            pltpu.SemaphoreType.DMA((KX,)),
            pltpu.SemaphoreType.DMA((KX,)),
            pltpu.SemaphoreType.DMA((KZ,)),
            pltpu.SemaphoreType.DMA((KZ,)),
        ],
        compiler_params=pltpu.CompilerParams(collective_id=0),
    )(x)
